# baseline (speedup 1.0000x reference)
import functools

import jax
import jax.numpy as jnp
from jax import lax
from jax.experimental import pallas as pl
from jax.experimental.pallas import tpu as pltpu
from jax.experimental.pallas import tpu_sc as plsc


def _build_sc_gather(B, D, M, A):
    info = plsc.get_sparse_core_info()
    NC, NS = info.num_cores, info.num_subcores
    NW = NC * NS
    CHUNK = B // NW
    NGRP = CHUNK // 128
    pw = [pow(31, i, M) for i in range(D)]

    mesh = plsc.VectorSubcoreMesh(core_axis_name="c", subcore_axis_name="s")

    @functools.partial(
        pl.kernel,
        out_type=[
            jax.ShapeDtypeStruct((A, B), jnp.float32),
            jax.ShapeDtypeStruct((A, B), jnp.float32),
        ],
        mesh=mesh,
        compiler_params=pltpu.CompilerParams(
            use_tc_tiling_on_sc=False, needs_layout_passes=False),
        scratch_types=[
            pltpu.VMEM((D, CHUNK), jnp.int32),
            pltpu.VMEM((NGRP, 128), jnp.int32),
            pltpu.VMEM((A, 256), jnp.float32),
            pltpu.VMEM((A, 256), jnp.float32),
            pltpu.VMEM((A, CHUNK), jnp.float32),
            pltpu.VMEM((A, CHUNK), jnp.float32),
            pltpu.SemaphoreType.DMA,
        ],
    )
    def sc_gather(xt_hbm, qt_hbm, pt_hbm, qout_hbm, pout_hbm,
                  xv, hidx, sq, sp, qv, pv, sem):
        wid = lax.axis_index("s") * NC + lax.axis_index("c")
        base = wid * CHUNK
        pltpu.sync_copy(xt_hbm.at[:, pl.ds(base, CHUNK)], xv)
        for g in range(CHUNK // 16):
            j, off = g // 8, (g % 8) * 16
            acc = xv[0, pl.ds(g * 16, 16)] * pw[0]
            for d in range(1, D):
                acc = acc + xv[d, pl.ds(g * 16, 16)] * pw[d]
            hidx[j, pl.ds(off, 16)] = lax.rem(acc, M)

        lanes = lax.iota(jnp.int32, 16)

        def body(g, _):
            j = lax.shift_right_logical(g, 3)
            off = lax.bitwise_and(g, 7) * 16
            v = hidx[j, pl.ds(off, 16)]
            copies = []
            for lane in range(16):
                r0 = pl.multiple_of(lax.bitwise_and(v[lane], -16), 16)
                copies.append(pltpu.async_copy(
                    qt_hbm.at[:, pl.ds(r0, 16)],
                    sq.at[:, pl.ds(lane * 16, 16)], sem))
                copies.append(pltpu.async_copy(
                    pt_hbm.at[:, pl.ds(r0, 16)],
                    sp.at[:, pl.ds(lane * 16, 16)], sem))
            for c in copies:
                c.wait()
            src_col = lanes * 16 + lax.bitwise_and(v, 15)
            dst_col = g * 16 + lanes
            for c in range(A):
                cc = jnp.full((16,), c, jnp.int32)
                plsc.store_scatter(
                    qv, [cc, dst_col], plsc.load_gather(sq, [cc, src_col]))
                plsc.store_scatter(
                    pv, [cc, dst_col], plsc.load_gather(sp, [cc, src_col]))
            return 0

        lax.fori_loop(0, CHUNK // 16, body, 0)
        pltpu.sync_copy(qv, qout_hbm.at[:, pl.ds(base, CHUNK)])
        pltpu.sync_copy(pv, pout_hbm.at[:, pl.ds(base, CHUNK)])

    return sc_gather


def _dense_body(B, A, qt_ref, at_ref, vals_ref, vidx_ref, probst_ref, ent_ref):
    q = qt_ref[...]
    a = at_ref[...]
    colmax = jnp.max(q, axis=0)
    vals_ref[...] = colmax
    act = lax.broadcasted_iota(jnp.int32, q.shape, 0)
    vidx_ref[...] = jnp.min(jnp.where(q == colmax[None, :], act, A), axis=0)
    am = jnp.max(a, axis=0, keepdims=True)
    s = a - am
    e = jnp.exp(s)
    z = jnp.sum(e, axis=0, keepdims=True)
    p = e / z
    probst_ref[...] = p
    lp = s - jnp.log(z)
    ent_blk = -jnp.sum(lp * p)

    @pl.when(pl.program_id(0) == 0)
    def _():
        ent_ref[...] = jnp.zeros((1, 1), jnp.float32)

    ent_ref[...] += jnp.full((1, 1), ent_blk / B, jnp.float32)


def _dense(qT, aT):
    A, B = qT.shape
    BLK = 2048
    grid = (B // BLK,)
    return pl.pallas_call(
        functools.partial(_dense_body, B, A),
        grid=grid,
        in_specs=[
            pl.BlockSpec((A, BLK), lambda i: (0, i)),
            pl.BlockSpec((A, BLK), lambda i: (0, i)),
        ],
        out_specs=[
            pl.BlockSpec((BLK,), lambda i: (i,)),
            pl.BlockSpec((BLK,), lambda i: (i,)),
            pl.BlockSpec((A, BLK), lambda i: (0, i)),
            pl.BlockSpec((1, 1), lambda i: (0, 0)),
        ],
        out_shape=[
            jax.ShapeDtypeStruct((B,), jnp.float32),
            jax.ShapeDtypeStruct((B,), jnp.int32),
            jax.ShapeDtypeStruct((A, B), jnp.float32),
            jax.ShapeDtypeStruct((1, 1), jnp.float32),
        ],
    )(qT, aT)


def kernel(x, Qtable, aprob_table):
    B, D = x.shape
    M, A = Qtable.shape
    sc_gather = _build_sc_gather(B, D, M, A)
    qT, pT = sc_gather(x.T, Qtable.T, aprob_table.T)
    values, vidx, probsT, ent = _dense(qT, pT)
    return (values, vidx, ent.reshape(()), probsT.T, qT.T)


# speedup vs baseline: 8.7790x; 8.7790x over previous
import functools

import jax
import jax.numpy as jnp
from jax import lax
from jax.experimental import pallas as pl
from jax.experimental.pallas import tpu as pltpu
from jax.experimental.pallas import tpu_sc as plsc


def _build_sc_gather(B, D, M, A):
    info = plsc.get_sparse_core_info()
    NC, NS = info.num_cores, info.num_subcores
    NW = NC * NS
    CHUNK = B // NW
    NGRP = CHUNK // 128
    HALF = CHUNK // 2
    pw = [pow(31, i, M) for i in range(D)]

    mesh = plsc.VectorSubcoreMesh(core_axis_name="c", subcore_axis_name="s")

    @functools.partial(
        pl.kernel,
        out_type=[
            jax.ShapeDtypeStruct((B, 2 * A), jnp.float32),
            jax.ShapeDtypeStruct((B, 2 * A), jnp.float32),
            jax.ShapeDtypeStruct((NW * NGRP, 128), jnp.int32),
        ],
        mesh=mesh,
        compiler_params=pltpu.CompilerParams(use_tc_tiling_on_sc=False),
        scratch_types=[
            pltpu.VMEM((D, CHUNK), jnp.int32),
            pltpu.VMEM((NGRP, 128), jnp.int32),
            pltpu.VMEM((NGRP, 128), jnp.int32),
            pltpu.VMEM((HALF, 2 * A), jnp.float32),
            pltpu.VMEM((HALF, 2 * A), jnp.float32),
            pltpu.SemaphoreType.DMA,
        ],
    )
    def sc_gather(xt_hbm, qp_hbm, pp_hbm, qout_hbm, aout_hbm, iout_hbm,
                  xv, hidx, pidx, qpv, apv, sem):
        wid = lax.axis_index("s") * NC + lax.axis_index("c")
        base = wid * CHUNK
        pltpu.sync_copy(xt_hbm.at[:, pl.ds(base, CHUNK)], xv)
        for g in range(CHUNK // 16):
            j, off = g // 8, (g % 8) * 16
            acc = xv[0, pl.ds(g * 16, 16)] * pw[0]
            for d in range(1, D):
                acc = acc + xv[d, pl.ds(g * 16, 16)] * pw[d]
            idx16 = lax.rem(acc, M)
            hidx[j, pl.ds(off, 16)] = idx16
            pidx[j, pl.ds(off, 16)] = lax.shift_right_logical(idx16, 1)
        for h in range(2):
            copies = []
            for jj in range(2):
                j = 2 * h + jj
                copies.append(pltpu.async_copy(
                    qp_hbm.at[pidx.at[j]], qpv.at[pl.ds(jj * 128, 128)], sem))
                copies.append(pltpu.async_copy(
                    pp_hbm.at[pidx.at[j]], apv.at[pl.ds(jj * 128, 128)], sem))
            for c in copies:
                c.wait()
            pltpu.sync_copy(qpv, qout_hbm.at[pl.ds(base + h * HALF, HALF)])
            pltpu.sync_copy(apv, aout_hbm.at[pl.ds(base + h * HALF, HALF)])
        pltpu.sync_copy(hidx, iout_hbm.at[pl.ds(wid * NGRP, NGRP)])

    return sc_gather


def _dense_body(B, A, qp_ref, ap_ref, idx_ref, vals_ref, vidx_ref,
                probs_ref, qvals_ref, ent_ref):
    qp = qp_ref[...]
    ap = ap_ref[...]
    bit = (idx_ref[...] & 1)[:, None]
    q = jnp.where(bit == 1, qp[:, A:], qp[:, :A])
    a = jnp.where(bit == 1, ap[:, A:], ap[:, :A])
    qvals_ref[...] = q
    rowmax = jnp.max(q, axis=1)
    vals_ref[...] = rowmax
    act = lax.broadcasted_iota(jnp.int32, q.shape, 1)
    vidx_ref[...] = jnp.min(jnp.where(q == rowmax[:, None], act, A), axis=1)
    am = jnp.max(a, axis=1, keepdims=True)
    s = a - am
    e = jnp.exp(s)
    z = jnp.sum(e, axis=1, keepdims=True)
    p = e / z
    probs_ref[...] = p
    lp = s - jnp.log(z)
    ent_blk = -jnp.sum(lp * p)

    @pl.when(pl.program_id(0) == 0)
    def _():
        ent_ref[...] = jnp.zeros((1, 1), jnp.float32)

    ent_ref[...] += jnp.full((1, 1), ent_blk / B, jnp.float32)


def _dense(qpk, apk, idx1):
    B = qpk.shape[0]
    A = qpk.shape[1] // 2
    BLK = 2048
    grid = (B // BLK,)
    return pl.pallas_call(
        functools.partial(_dense_body, B, A),
        grid=grid,
        in_specs=[
            pl.BlockSpec((BLK, 2 * A), lambda i: (i, 0)),
            pl.BlockSpec((BLK, 2 * A), lambda i: (i, 0)),
            pl.BlockSpec((BLK,), lambda i: (i,)),
        ],
        out_specs=[
            pl.BlockSpec((BLK,), lambda i: (i,)),
            pl.BlockSpec((BLK,), lambda i: (i,)),
            pl.BlockSpec((BLK, A), lambda i: (i, 0)),
            pl.BlockSpec((BLK, A), lambda i: (i, 0)),
            pl.BlockSpec((1, 1), lambda i: (0, 0)),
        ],
        out_shape=[
            jax.ShapeDtypeStruct((B,), jnp.float32),
            jax.ShapeDtypeStruct((B,), jnp.int32),
            jax.ShapeDtypeStruct((B, A), jnp.float32),
            jax.ShapeDtypeStruct((B, A), jnp.float32),
            jax.ShapeDtypeStruct((1, 1), jnp.float32),
        ],
    )(qpk, apk, idx1)


def kernel(x, Qtable, aprob_table):
    B, D = x.shape
    M, A = Qtable.shape
    qp = Qtable.reshape(M // 2, 2 * A)
    pp = aprob_table.reshape(M // 2, 2 * A)
    sc_gather = _build_sc_gather(B, D, M, A)
    qpk, apk, idx2 = sc_gather(x.T, qp, pp)
    values, vidx, probs, qvals, ent = _dense(qpk, apk, idx2.reshape(B))
    return (values, vidx, ent.reshape(()), probs, qvals)


# speedup vs baseline: 9.1657x; 1.0441x over previous
import functools

import jax
import jax.numpy as jnp
from jax import lax
from jax.experimental import pallas as pl
from jax.experimental.pallas import tpu as pltpu
from jax.experimental.pallas import tpu_sc as plsc


def _build_sc_gather(B, D, M, A):
    info = plsc.get_sparse_core_info()
    NC, NS = info.num_cores, info.num_subcores
    NW = NC * NS
    CHUNK = B // NW
    NGRP = CHUNK // 128
    HALF = CHUNK // 2
    pw = [pow(31, i, M) for i in range(D)]

    mesh = plsc.VectorSubcoreMesh(core_axis_name="c", subcore_axis_name="s")

    @functools.partial(
        pl.kernel,
        out_type=[
            jax.ShapeDtypeStruct((B, 2 * A), jnp.float32),
            jax.ShapeDtypeStruct((B, 2 * A), jnp.float32),
            jax.ShapeDtypeStruct((NW * NGRP, 128), jnp.int32),
        ],
        mesh=mesh,
        compiler_params=pltpu.CompilerParams(use_tc_tiling_on_sc=False),
        scratch_types=[
            pltpu.VMEM((D, CHUNK), jnp.int32),
            pltpu.VMEM((NGRP, 128), jnp.int32),
            pltpu.VMEM((NGRP, 128), jnp.int32),
            pltpu.VMEM((128, 4 * A), jnp.float32),
            pltpu.VMEM((128, 4 * A), jnp.float32),
            pltpu.SemaphoreType.DMA,
        ],
    )
    def sc_gather(xt_hbm, qp_hbm, pp_hbm, qout_hbm, aout_hbm, iout_hbm,
                  xv, hidx, pidx, qpv, apv, sem):
        wid = lax.axis_index("s") * NC + lax.axis_index("c")
        base = wid * CHUNK
        pltpu.sync_copy(xt_hbm.at[:, pl.ds(base, CHUNK)], xv)
        for g in range(CHUNK // 16):
            j, off = g // 8, (g % 8) * 16
            acc = xv[0, pl.ds(g * 16, 16)] * pw[0]
            for d in range(1, D):
                acc = acc + xv[d, pl.ds(g * 16, 16)] * pw[d]
            idx16 = lax.rem(acc, M)
            hidx[j, pl.ds(off, 16)] = idx16
            pidx[j, pl.ds(off, 16)] = lax.shift_right_logical(idx16, 1)
        for j in range(NGRP):
            c1 = pltpu.async_copy(qp_hbm.at[pidx.at[j]], qpv, sem)
            c2 = pltpu.async_copy(pp_hbm.at[pidx.at[j]], apv, sem)
            c1.wait()
            c2.wait()
            rows = base + j * 128
            pltpu.sync_copy(qpv.at[:, pl.ds(0, A)],
                            qout_hbm.at[pl.ds(rows, 128), pl.ds(0, A)])
            pltpu.sync_copy(qpv.at[:, pl.ds(2 * A, A)],
                            qout_hbm.at[pl.ds(rows, 128), pl.ds(A, A)])
            pltpu.sync_copy(apv.at[:, pl.ds(0, A)],
                            aout_hbm.at[pl.ds(rows, 128), pl.ds(0, A)])
            pltpu.sync_copy(apv.at[:, pl.ds(2 * A, A)],
                            aout_hbm.at[pl.ds(rows, 128), pl.ds(A, A)])
        pltpu.sync_copy(hidx, iout_hbm.at[pl.ds(wid * NGRP, NGRP)])

    return sc_gather


def _dense_body(B, A, qp_ref, ap_ref, idx_ref, vals_ref, vidx_ref,
                probs_ref, qvals_ref, ent_ref):
    qp = qp_ref[...]
    ap = ap_ref[...]
    bit = (idx_ref[...] & 1)[:, None]
    q = jnp.where(bit == 1, qp[:, A:], qp[:, :A])
    a = jnp.where(bit == 1, ap[:, A:], ap[:, :A])
    qvals_ref[...] = q
    rowmax = jnp.max(q, axis=1)
    vals_ref[...] = rowmax
    act = lax.broadcasted_iota(jnp.int32, q.shape, 1)
    vidx_ref[...] = jnp.min(jnp.where(q == rowmax[:, None], act, A), axis=1)
    am = jnp.max(a, axis=1, keepdims=True)
    s = a - am
    e = jnp.exp(s)
    z = jnp.sum(e, axis=1, keepdims=True)
    p = e / z
    probs_ref[...] = p
    lp = s - jnp.log(z)
    ent_blk = -jnp.sum(lp * p)

    @pl.when(pl.program_id(0) == 0)
    def _():
        ent_ref[...] = jnp.zeros((1, 1), jnp.float32)

    ent_ref[...] += jnp.full((1, 1), ent_blk / B, jnp.float32)


def _dense(qpk, apk, idx1):
    B = qpk.shape[0]
    A = qpk.shape[1] // 2
    BLK = 2048
    grid = (B // BLK,)
    return pl.pallas_call(
        functools.partial(_dense_body, B, A),
        grid=grid,
        in_specs=[
            pl.BlockSpec((BLK, 2 * A), lambda i: (i, 0)),
            pl.BlockSpec((BLK, 2 * A), lambda i: (i, 0)),
            pl.BlockSpec((BLK,), lambda i: (i,)),
        ],
        out_specs=[
            pl.BlockSpec((BLK,), lambda i: (i,)),
            pl.BlockSpec((BLK,), lambda i: (i,)),
            pl.BlockSpec((BLK, A), lambda i: (i, 0)),
            pl.BlockSpec((BLK, A), lambda i: (i, 0)),
            pl.BlockSpec((1, 1), lambda i: (0, 0)),
        ],
        out_shape=[
            jax.ShapeDtypeStruct((B,), jnp.float32),
            jax.ShapeDtypeStruct((B,), jnp.int32),
            jax.ShapeDtypeStruct((B, A), jnp.float32),
            jax.ShapeDtypeStruct((B, A), jnp.float32),
            jax.ShapeDtypeStruct((1, 1), jnp.float32),
        ],
    )(qpk, apk, idx1)


def kernel(x, Qtable, aprob_table):
    B, D = x.shape
    M, A = Qtable.shape
    qp = jnp.pad(Qtable.reshape(M // 2, 2, A),
                 ((0, 0), (0, 0), (0, A))).reshape(M // 2, 4 * A)
    pp = jnp.pad(aprob_table.reshape(M // 2, 2, A),
                 ((0, 0), (0, 0), (0, A))).reshape(M // 2, 4 * A)
    sc_gather = _build_sc_gather(B, D, M, A)
    qpk, apk, idx2 = sc_gather(x.T, qp, pp)
    values, vidx, probs, qvals, ent = _dense(qpk, apk, idx2.reshape(B))
    return (values, vidx, ent.reshape(()), probs, qvals)
